# baseline (speedup 1.0000x reference)
import functools

import jax
import jax.numpy as jnp
from jax import lax
from jax.experimental import pallas as pl
from jax.experimental.pallas import tpu as pltpu
from jax.experimental.pallas import tpu_sc as plsc

N = 10000
E = 160000
DIN = 256
DH = 256
DOUT = 128
DC = 128

NB = 1000
NBLK = N // NB
EC = 125
EROWS = E // EC

RMAIN = 624
RTAIL = N - 16 * RMAIN
ZR = 104

_MESH = plsc.VectorSubcoreMesh(core_axis_name="c", subcore_axis_name="s")


@functools.partial(
    pl.kernel,
    mesh=_MESH,
    out_type=jax.ShapeDtypeStruct((N,), jnp.float32),
    scratch_types=[
        pltpu.VMEM((EROWS // 16, EC), jnp.int32),
        pltpu.VMEM((640,), jnp.float32),
        pltpu.VMEM((128,), jnp.float32),
        pltpu.VMEM_SHARED((N,), jnp.float32),
    ],
)
def _deg_kernel(dst_hbm, out_hbm, idx_v, zv, ones_v, acc):
    c = lax.axis_index("c")
    s = lax.axis_index("s")

    @pl.when(c == 0)
    def _():
        @pl.loop(0, 640, step=16)
        def _(i):
            zv[pl.ds(i, 16)] = jnp.zeros((16,), jnp.float32)

        @pl.loop(0, 128, step=16)
        def _(i):
            ones_v[pl.ds(i, 16)] = jnp.full((16,), 1.0, jnp.float32)

        @pl.when(s == 0)
        def _():
            for k in range(15):
                pltpu.sync_copy(zv, acc.at[pl.ds(k * 640, 640)])
            pltpu.sync_copy(zv.at[pl.ds(0, 400)], acc.at[pl.ds(9600, 400)])

        plsc.subcore_barrier()
        nrows = EROWS // 16
        pltpu.sync_copy(dst_hbm.at[pl.ds(s * nrows, nrows)], idx_v)

        @pl.loop(0, nrows)
        def _(j):
            pltpu.sync_copy(ones_v.at[pl.ds(0, EC)], acc.at[idx_v.at[j]],
                            add=True)

        plsc.subcore_barrier()

        @pl.when(s == 0)
        def _():
            pltpu.sync_copy(acc, out_hbm)


def _make_agg(feature_split):
    nrows = (EROWS // 16) if feature_split else (EROWS // 32)
    table_shape = (2, N, DC) if feature_split else (N, DC)
    PH = 40
    nphase = nrows // PH

    @functools.partial(
        pl.kernel,
        mesh=_MESH,
        out_type=jax.ShapeDtypeStruct((2, N, DC), jnp.float32),
        scratch_types=[
            pltpu.VMEM((PH, EC), jnp.int32),
            pltpu.VMEM((PH, EC), jnp.int32),
            pltpu.VMEM((EC, DC), jnp.float32),
            pltpu.VMEM((EC, DC), jnp.float32),
            pltpu.VMEM_SHARED((N, DC), jnp.float32),
            pltpu.SemaphoreType.DMA,
            pltpu.SemaphoreType.DMA,
            pltpu.SemaphoreType.DMA,
            pltpu.SemaphoreType.DMA,
        ],
    )
    def agg(h_hbm, src_hbm, dst_hbm, out_hbm, isrc, idst, rowbuf, rowbuf2,
            acc, sem, sem2, ssem, ssem2):
        c = lax.axis_index("c")
        s = lax.axis_index("s")

        @pl.loop(0, EC)
        def _(i):
            @pl.loop(0, DC, step=16)
            def _(j):
                rowbuf[i, pl.ds(j, 16)] = jnp.zeros((16,), jnp.float32)

        for k in range(RMAIN // ZR):
            pltpu.sync_copy(rowbuf.at[pl.ds(0, ZR)],
                            acc.at[pl.ds(s * RMAIN + k * ZR, ZR)])

        @pl.when(s == 0)
        def _():
            pltpu.sync_copy(rowbuf.at[pl.ds(0, RTAIL)],
                            acc.at[pl.ds(16 * RMAIN, RTAIL)])

        plsc.subcore_barrier()

        row0 = s * nrows if feature_split else (c * (EROWS // 2) + s * nrows)
        table = h_hbm.at[c] if feature_split else h_hbm

        for p in range(nphase):
            pltpu.sync_copy(src_hbm.at[pl.ds(row0 + p * PH, PH)], isrc)
            pltpu.sync_copy(dst_hbm.at[pl.ds(row0 + p * PH, PH)], idst)

            pltpu.async_copy(table.at[isrc.at[0]], rowbuf, sem)
            pltpu.async_copy(table.at[isrc.at[1]], rowbuf2, sem2)

            @pl.loop(0, PH, step=2)
            def _(j):
                pltpu.make_async_copy(table.at[isrc.at[j]], rowbuf,
                                      sem).wait()
                pltpu.async_copy(rowbuf, acc.at[idst.at[j]], ssem, add=True)
                pltpu.make_async_copy(table.at[isrc.at[j + 1]], rowbuf2,
                                      sem2).wait()
                pltpu.async_copy(rowbuf2, acc.at[idst.at[j + 1]], ssem2,
                                 add=True)
                pltpu.make_async_copy(rowbuf, acc.at[idst.at[j]],
                                      ssem).wait()

                @pl.when(j + 2 < PH)
                def _():
                    pltpu.async_copy(table.at[isrc.at[j + 2]], rowbuf, sem)

                pltpu.make_async_copy(rowbuf2, acc.at[idst.at[j + 1]],
                                      ssem2).wait()

                @pl.when(j + 3 < PH)
                def _():
                    pltpu.async_copy(table.at[isrc.at[j + 3]], rowbuf2, sem2)

        plsc.subcore_barrier()
        for k in range(RMAIN // ZR):
            r0 = s * RMAIN + k * ZR
            pltpu.sync_copy(acc.at[pl.ds(r0, ZR)], rowbuf.at[pl.ds(0, ZR)])
            pltpu.sync_copy(rowbuf.at[pl.ds(0, ZR)],
                            out_hbm.at[c].at[pl.ds(r0, ZR)])

        @pl.when(s == 0)
        def _():
            pltpu.sync_copy(acc.at[pl.ds(16 * RMAIN, RTAIL)],
                            rowbuf.at[pl.ds(0, RTAIL)])
            pltpu.sync_copy(rowbuf.at[pl.ds(0, RTAIL)],
                            out_hbm.at[c].at[pl.ds(16 * RMAIN, RTAIL)])

    def wrapped(h, src2, dst2):
        assert h.shape == table_shape
        return agg(h, src2, dst2)

    return wrapped


_agg_l1 = _make_agg(feature_split=True)
_agg_l2 = _make_agg(feature_split=False)


def _mm1_body(x_ref, w_ref, out_ref):
    out_ref[0] = jnp.dot(x_ref[...], w_ref[...],
                         preferred_element_type=jnp.float32)


def _mm1(x, w1):
    return pl.pallas_call(
        _mm1_body,
        grid=(2, NBLK),
        in_specs=[
            pl.BlockSpec((NB, DIN), lambda j, i: (i, 0)),
            pl.BlockSpec((DIN, DH // 2), lambda j, i: (0, j)),
        ],
        out_specs=pl.BlockSpec((1, NB, DH // 2), lambda j, i: (j, i, 0)),
        out_shape=jax.ShapeDtypeStruct((2, N, DH // 2), jnp.float32),
    )(x, w1)


def _scale_body(h_ref, d_ref, out_ref):
    out_ref[...] = h_ref[...] * lax.rsqrt(d_ref[...] + 1.0)[None]


def _scale(h1, draw2):
    return pl.pallas_call(
        _scale_body,
        grid=(NBLK,),
        in_specs=[
            pl.BlockSpec((2, NB, DH // 2), lambda i: (0, i, 0)),
            pl.BlockSpec((NB, 1), lambda i: (i, 0)),
        ],
        out_specs=pl.BlockSpec((2, NB, DH // 2), lambda i: (0, i, 0)),
        out_shape=jax.ShapeDtypeStruct((2, N, DH // 2), jnp.float32),
    )(h1, draw2)


def _mid_body(g_ref, hs_ref, d_ref, b_ref, w_ref, out_ref):
    scale = lax.rsqrt(d_ref[...] + 1.0)
    a = jax.nn.relu((g_ref[...] + hs_ref[...]) * scale[None]
                    + b_ref[...].reshape(2, 1, DH // 2))
    h2 = (jnp.dot(a[0], w_ref[0], preferred_element_type=jnp.float32)
          + jnp.dot(a[1], w_ref[1], preferred_element_type=jnp.float32))
    out_ref[...] = h2 * scale


def _mid(g1, hs1, draw2, b1r, w2r):
    return pl.pallas_call(
        _mid_body,
        grid=(NBLK,),
        in_specs=[
            pl.BlockSpec((2, NB, DH // 2), lambda i: (0, i, 0)),
            pl.BlockSpec((2, NB, DH // 2), lambda i: (0, i, 0)),
            pl.BlockSpec((NB, 1), lambda i: (i, 0)),
            pl.BlockSpec((2, DH // 2), lambda i: (0, 0)),
            pl.BlockSpec((2, DH // 2, DOUT), lambda i: (0, 0, 0)),
        ],
        out_specs=pl.BlockSpec((NB, DOUT), lambda i: (i, 0)),
        out_shape=jax.ShapeDtypeStruct((N, DOUT), jnp.float32),
    )(g1, hs1, draw2, b1r, w2r)


def _out_body(g_ref, hs_ref, d_ref, b_ref, out_ref):
    scale = lax.rsqrt(d_ref[...] + 1.0)
    zf = (g_ref[0] + g_ref[1] + hs_ref[...]) * scale + b_ref[...]
    m = jnp.max(zf, axis=-1, keepdims=True)
    lse = jnp.log(jnp.sum(jnp.exp(zf - m), axis=-1, keepdims=True)) + m
    out_ref[...] = zf - lse


def _outk(g2, hs2, draw2, b2r):
    return pl.pallas_call(
        _out_body,
        grid=(NBLK,),
        in_specs=[
            pl.BlockSpec((2, NB, DOUT), lambda i: (0, i, 0)),
            pl.BlockSpec((NB, DOUT), lambda i: (i, 0)),
            pl.BlockSpec((NB, 1), lambda i: (i, 0)),
            pl.BlockSpec((1, DOUT), lambda i: (0, 0)),
        ],
        out_specs=pl.BlockSpec((NB, DOUT), lambda i: (i, 0)),
        out_shape=jax.ShapeDtypeStruct((N, DOUT), jnp.float32),
    )(g2, hs2, draw2, b2r)


def kernel(x, edge_index, W1, b1, W2, b2):
    src2 = edge_index[0].reshape(EROWS, EC)
    dst2 = edge_index[1].reshape(EROWS, EC)

    draw = _deg_kernel(dst2)
    draw2 = draw.reshape(N, 1)

    h1 = _mm1(x, W1)
    hs1 = _scale(h1, draw2)
    g1 = _agg_l1(hs1, src2, dst2)
    hs2 = _mid(g1, hs1, draw2, b1.reshape(2, DH // 2),
               W2.reshape(2, DH // 2, DOUT))
    g2 = _agg_l2(hs2, src2, dst2)
    return _outk(g2, hs2, draw2, b2.reshape(1, DOUT))


# speedup vs baseline: 1.2001x; 1.2001x over previous
import functools

import jax
import jax.numpy as jnp
from jax import lax
from jax.experimental import pallas as pl
from jax.experimental.pallas import tpu as pltpu
from jax.experimental.pallas import tpu_sc as plsc

N = 10000
E = 160000
DIN = 256
DH = 256
DOUT = 128
DC = 128

NB = 1000
NBLK = N // NB
EC = 125
EROWS = E // EC

RMAIN = 624
RTAIL = N - 16 * RMAIN
ZR = 104

_MESH = plsc.VectorSubcoreMesh(core_axis_name="c", subcore_axis_name="s")


@functools.partial(
    pl.kernel,
    mesh=_MESH,
    out_type=jax.ShapeDtypeStruct((N,), jnp.float32),
    scratch_types=[
        pltpu.VMEM((EROWS // 16, EC), jnp.int32),
        pltpu.VMEM((640,), jnp.float32),
        pltpu.VMEM((128,), jnp.float32),
        pltpu.VMEM_SHARED((N,), jnp.float32),
    ],
)
def _deg_kernel(dst_hbm, out_hbm, idx_v, zv, ones_v, acc):
    c = lax.axis_index("c")
    s = lax.axis_index("s")

    @pl.when(c == 0)
    def _():
        @pl.loop(0, 640, step=16)
        def _(i):
            zv[pl.ds(i, 16)] = jnp.zeros((16,), jnp.float32)

        @pl.loop(0, 128, step=16)
        def _(i):
            ones_v[pl.ds(i, 16)] = jnp.full((16,), 1.0, jnp.float32)

        @pl.when(s == 0)
        def _():
            for k in range(15):
                pltpu.sync_copy(zv, acc.at[pl.ds(k * 640, 640)])
            pltpu.sync_copy(zv.at[pl.ds(0, 400)], acc.at[pl.ds(9600, 400)])

        plsc.subcore_barrier()
        nrows = EROWS // 16
        pltpu.sync_copy(dst_hbm.at[pl.ds(s * nrows, nrows)], idx_v)

        @pl.loop(0, nrows)
        def _(j):
            pltpu.sync_copy(ones_v.at[pl.ds(0, EC)], acc.at[idx_v.at[j]],
                            add=True)

        plsc.subcore_barrier()

        @pl.when(s == 0)
        def _():
            pltpu.sync_copy(acc, out_hbm)


def _make_agg(feature_split):
    nrows = (EROWS // 16) if feature_split else (EROWS // 32)
    table_shape = (2, N, DC) if feature_split else (N, DC)
    PH = 40
    nphase = nrows // PH

    @functools.partial(
        pl.kernel,
        mesh=_MESH,
        out_type=jax.ShapeDtypeStruct((2, N, DC), jnp.float32),
        scratch_types=[
            pltpu.VMEM((PH, EC), jnp.int32),
            pltpu.VMEM((PH, EC), jnp.int32),
            pltpu.VMEM((EC, DC), jnp.float32),
            pltpu.VMEM((EC, DC), jnp.float32),
            pltpu.VMEM_SHARED((N, DC), jnp.float32),
            pltpu.SemaphoreType.DMA,
            pltpu.SemaphoreType.DMA,
            pltpu.SemaphoreType.DMA,
            pltpu.SemaphoreType.DMA,
        ],
    )
    def agg(h_hbm, src_hbm, dst_hbm, out_hbm, isrc, idst, rowbuf, rowbuf2,
            acc, sem, sem2, ssem, ssem2):
        c = lax.axis_index("c")
        s = lax.axis_index("s")

        @pl.loop(0, EC)
        def _(i):
            @pl.loop(0, DC, step=16)
            def _(j):
                rowbuf[i, pl.ds(j, 16)] = jnp.zeros((16,), jnp.float32)

        for k in range(RMAIN // ZR):
            pltpu.sync_copy(rowbuf.at[pl.ds(0, ZR)],
                            acc.at[pl.ds(s * RMAIN + k * ZR, ZR)])

        @pl.when(s == 0)
        def _():
            pltpu.sync_copy(rowbuf.at[pl.ds(0, RTAIL)],
                            acc.at[pl.ds(16 * RMAIN, RTAIL)])

        plsc.subcore_barrier()

        row0 = s * nrows if feature_split else (c * (EROWS // 2) + s * nrows)
        table = h_hbm.at[c] if feature_split else h_hbm

        for p in range(nphase):
            pltpu.sync_copy(src_hbm.at[pl.ds(row0 + p * PH, PH)], isrc)
            pltpu.sync_copy(dst_hbm.at[pl.ds(row0 + p * PH, PH)], idst)

            pltpu.async_copy(table.at[isrc.at[0]], rowbuf, sem)
            pltpu.async_copy(table.at[isrc.at[1]], rowbuf2, sem2)

            @pl.loop(0, PH, step=2)
            def _(j):
                pltpu.make_async_copy(table.at[isrc.at[j]], rowbuf,
                                      sem).wait()
                pltpu.sync_copy(rowbuf, acc.at[idst.at[j]], add=True)

                @pl.when(j + 2 < PH)
                def _():
                    pltpu.async_copy(table.at[isrc.at[j + 2]], rowbuf, sem)

                pltpu.make_async_copy(table.at[isrc.at[j + 1]], rowbuf2,
                                      sem2).wait()
                pltpu.sync_copy(rowbuf2, acc.at[idst.at[j + 1]], add=True)

                @pl.when(j + 3 < PH)
                def _():
                    pltpu.async_copy(table.at[isrc.at[j + 3]], rowbuf2, sem2)

        plsc.subcore_barrier()
        for k in range(RMAIN // ZR):
            r0 = s * RMAIN + k * ZR
            pltpu.sync_copy(acc.at[pl.ds(r0, ZR)], rowbuf.at[pl.ds(0, ZR)])
            pltpu.sync_copy(rowbuf.at[pl.ds(0, ZR)],
                            out_hbm.at[c].at[pl.ds(r0, ZR)])

        @pl.when(s == 0)
        def _():
            pltpu.sync_copy(acc.at[pl.ds(16 * RMAIN, RTAIL)],
                            rowbuf.at[pl.ds(0, RTAIL)])
            pltpu.sync_copy(rowbuf.at[pl.ds(0, RTAIL)],
                            out_hbm.at[c].at[pl.ds(16 * RMAIN, RTAIL)])

    def wrapped(h, src2, dst2):
        assert h.shape == table_shape
        return agg(h, src2, dst2)

    return wrapped


_agg_l1 = _make_agg(feature_split=True)
_agg_l2 = _make_agg(feature_split=False)


def _mm1_body(x_ref, w_ref, d_ref, out_ref):
    h = jnp.dot(x_ref[...], w_ref[...], preferred_element_type=jnp.float32)
    out_ref[0] = h * lax.rsqrt(d_ref[...] + 1.0)


def _mm1(x, w1, draw2):
    return pl.pallas_call(
        _mm1_body,
        grid=(2, NBLK),
        in_specs=[
            pl.BlockSpec((NB, DIN), lambda j, i: (i, 0)),
            pl.BlockSpec((DIN, DH // 2), lambda j, i: (0, j)),
            pl.BlockSpec((NB, 1), lambda j, i: (i, 0)),
        ],
        out_specs=pl.BlockSpec((1, NB, DH // 2), lambda j, i: (j, i, 0)),
        out_shape=jax.ShapeDtypeStruct((2, N, DH // 2), jnp.float32),
    )(x, w1, draw2)


def _mid_body(g_ref, hs_ref, d_ref, b_ref, w_ref, out_ref):
    scale = lax.rsqrt(d_ref[...] + 1.0)
    a = jax.nn.relu((g_ref[...] + hs_ref[...]) * scale[None]
                    + b_ref[...].reshape(2, 1, DH // 2))
    h2 = (jnp.dot(a[0], w_ref[0], preferred_element_type=jnp.float32)
          + jnp.dot(a[1], w_ref[1], preferred_element_type=jnp.float32))
    out_ref[...] = h2 * scale


def _mid(g1, hs1, draw2, b1r, w2r):
    return pl.pallas_call(
        _mid_body,
        grid=(NBLK,),
        in_specs=[
            pl.BlockSpec((2, NB, DH // 2), lambda i: (0, i, 0)),
            pl.BlockSpec((2, NB, DH // 2), lambda i: (0, i, 0)),
            pl.BlockSpec((NB, 1), lambda i: (i, 0)),
            pl.BlockSpec((2, DH // 2), lambda i: (0, 0)),
            pl.BlockSpec((2, DH // 2, DOUT), lambda i: (0, 0, 0)),
        ],
        out_specs=pl.BlockSpec((NB, DOUT), lambda i: (i, 0)),
        out_shape=jax.ShapeDtypeStruct((N, DOUT), jnp.float32),
    )(g1, hs1, draw2, b1r, w2r)


def _out_body(g_ref, hs_ref, d_ref, b_ref, out_ref):
    scale = lax.rsqrt(d_ref[...] + 1.0)
    zf = (g_ref[0] + g_ref[1] + hs_ref[...]) * scale + b_ref[...]
    m = jnp.max(zf, axis=-1, keepdims=True)
    lse = jnp.log(jnp.sum(jnp.exp(zf - m), axis=-1, keepdims=True)) + m
    out_ref[...] = zf - lse


def _outk(g2, hs2, draw2, b2r):
    return pl.pallas_call(
        _out_body,
        grid=(NBLK,),
        in_specs=[
            pl.BlockSpec((2, NB, DOUT), lambda i: (0, i, 0)),
            pl.BlockSpec((NB, DOUT), lambda i: (i, 0)),
            pl.BlockSpec((NB, 1), lambda i: (i, 0)),
            pl.BlockSpec((1, DOUT), lambda i: (0, 0)),
        ],
        out_specs=pl.BlockSpec((NB, DOUT), lambda i: (i, 0)),
        out_shape=jax.ShapeDtypeStruct((N, DOUT), jnp.float32),
    )(g2, hs2, draw2, b2r)


def kernel(x, edge_index, W1, b1, W2, b2):
    src2 = edge_index[0].reshape(EROWS, EC)
    dst2 = edge_index[1].reshape(EROWS, EC)

    draw = _deg_kernel(dst2)
    draw2 = draw.reshape(N, 1)

    hs1 = _mm1(x, W1, draw2)
    g1 = _agg_l1(hs1, src2, dst2)
    hs2 = _mid(g1, hs1, draw2, b1.reshape(2, DH // 2),
               W2.reshape(2, DH // 2, DOUT))
    g2 = _agg_l2(hs2, src2, dst2)
    return _outk(g2, hs2, draw2, b2.reshape(1, DOUT))


# speedup vs baseline: 1.2049x; 1.0040x over previous
import functools

import jax
import jax.numpy as jnp
from jax import lax
from jax.experimental import pallas as pl
from jax.experimental.pallas import tpu as pltpu
from jax.experimental.pallas import tpu_sc as plsc

N = 10000
E = 160000
DIN = 256
DH = 256
DOUT = 128
DC = 128

NB = 1000
NBLK = N // NB
EC = 125
EROWS = E // EC

RMAIN = 624
RTAIL = N - 16 * RMAIN
ZR = 104

_MESH = plsc.VectorSubcoreMesh(core_axis_name="c", subcore_axis_name="s")


@functools.partial(
    pl.kernel,
    mesh=_MESH,
    out_type=jax.ShapeDtypeStruct((2, N), jnp.float32),
    scratch_types=[
        pltpu.VMEM((EROWS // 32, EC), jnp.int32),
        pltpu.VMEM((640,), jnp.float32),
        pltpu.VMEM((128,), jnp.float32),
        pltpu.VMEM_SHARED((N,), jnp.float32),
    ],
)
def _deg_kernel(dst_hbm, out_hbm, idx_v, zv, ones_v, acc):
    c = lax.axis_index("c")
    s = lax.axis_index("s")

    @pl.loop(0, 640, step=16)
    def _(i):
        zv[pl.ds(i, 16)] = jnp.zeros((16,), jnp.float32)

    @pl.loop(0, 128, step=16)
    def _(i):
        ones_v[pl.ds(i, 16)] = jnp.full((16,), 1.0, jnp.float32)

    @pl.when(s == 0)
    def _():
        for k in range(15):
            pltpu.sync_copy(zv, acc.at[pl.ds(k * 640, 640)])
        pltpu.sync_copy(zv.at[pl.ds(0, 400)], acc.at[pl.ds(9600, 400)])

    plsc.subcore_barrier()
    nrows = EROWS // 32
    pltpu.sync_copy(dst_hbm.at[pl.ds(c * (EROWS // 2) + s * nrows, nrows)],
                    idx_v)

    @pl.loop(0, nrows)
    def _(j):
        pltpu.sync_copy(ones_v.at[pl.ds(0, EC)], acc.at[idx_v.at[j]],
                        add=True)

    plsc.subcore_barrier()

    @pl.when(s == 0)
    def _():
        pltpu.sync_copy(acc, out_hbm.at[c])


def _make_agg(feature_split):
    nrows = (EROWS // 16) if feature_split else (EROWS // 32)
    table_shape = (2, N, DC) if feature_split else (N, DC)
    PH = 40
    nphase = nrows // PH

    @functools.partial(
        pl.kernel,
        mesh=_MESH,
        out_type=jax.ShapeDtypeStruct((2, N, DC), jnp.float32),
        scratch_types=[
            pltpu.VMEM((PH, EC), jnp.int32),
            pltpu.VMEM((PH, EC), jnp.int32),
            pltpu.VMEM((EC, DC), jnp.float32),
            pltpu.VMEM((EC, DC), jnp.float32),
            pltpu.VMEM_SHARED((N, DC), jnp.float32),
            pltpu.SemaphoreType.DMA,
            pltpu.SemaphoreType.DMA,
            pltpu.SemaphoreType.DMA,
            pltpu.SemaphoreType.DMA,
        ],
    )
    def agg(h_hbm, src_hbm, dst_hbm, out_hbm, isrc, idst, rowbuf, rowbuf2,
            acc, sem, sem2, ssem, ssem2):
        c = lax.axis_index("c")
        s = lax.axis_index("s")

        @pl.loop(0, EC)
        def _(i):
            @pl.loop(0, DC, step=16)
            def _(j):
                rowbuf[i, pl.ds(j, 16)] = jnp.zeros((16,), jnp.float32)

        for k in range(RMAIN // ZR):
            pltpu.sync_copy(rowbuf.at[pl.ds(0, ZR)],
                            acc.at[pl.ds(s * RMAIN + k * ZR, ZR)])

        @pl.when(s == 0)
        def _():
            pltpu.sync_copy(rowbuf.at[pl.ds(0, RTAIL)],
                            acc.at[pl.ds(16 * RMAIN, RTAIL)])

        plsc.subcore_barrier()

        row0 = s * nrows if feature_split else (c * (EROWS // 2) + s * nrows)
        table = h_hbm.at[c] if feature_split else h_hbm

        for p in range(nphase):
            pltpu.sync_copy(src_hbm.at[pl.ds(row0 + p * PH, PH)], isrc)
            pltpu.sync_copy(dst_hbm.at[pl.ds(row0 + p * PH, PH)], idst)

            pltpu.async_copy(table.at[isrc.at[0]], rowbuf, sem)
            pltpu.async_copy(table.at[isrc.at[1]], rowbuf2, sem2)

            @pl.loop(0, PH, step=2)
            def _(j):
                pltpu.make_async_copy(table.at[isrc.at[j]], rowbuf,
                                      sem).wait()
                pltpu.sync_copy(rowbuf, acc.at[idst.at[j]], add=True)

                @pl.when(j + 2 < PH)
                def _():
                    pltpu.async_copy(table.at[isrc.at[j + 2]], rowbuf, sem)

                pltpu.make_async_copy(table.at[isrc.at[j + 1]], rowbuf2,
                                      sem2).wait()
                pltpu.sync_copy(rowbuf2, acc.at[idst.at[j + 1]], add=True)

                @pl.when(j + 3 < PH)
                def _():
                    pltpu.async_copy(table.at[isrc.at[j + 3]], rowbuf2, sem2)

        plsc.subcore_barrier()
        for k in range(RMAIN // ZR):
            r0 = s * RMAIN + k * ZR
            pltpu.sync_copy(acc.at[pl.ds(r0, ZR)], rowbuf.at[pl.ds(0, ZR)])
            pltpu.sync_copy(rowbuf.at[pl.ds(0, ZR)],
                            out_hbm.at[c].at[pl.ds(r0, ZR)])

        @pl.when(s == 0)
        def _():
            pltpu.sync_copy(acc.at[pl.ds(16 * RMAIN, RTAIL)],
                            rowbuf.at[pl.ds(0, RTAIL)])
            pltpu.sync_copy(rowbuf.at[pl.ds(0, RTAIL)],
                            out_hbm.at[c].at[pl.ds(16 * RMAIN, RTAIL)])

    def wrapped(h, src2, dst2):
        assert h.shape == table_shape
        return agg(h, src2, dst2)

    return wrapped


_agg_l1 = _make_agg(feature_split=True)
_agg_l2 = _make_agg(feature_split=False)


def _mm1_body(x_ref, w_ref, d_ref, out_ref):
    h = jnp.dot(x_ref[...], w_ref[...], preferred_element_type=jnp.float32)
    out_ref[0] = h * lax.rsqrt(d_ref[0] + d_ref[1] + 1.0)


def _mm1(x, w1, draw2):
    return pl.pallas_call(
        _mm1_body,
        grid=(2, NBLK),
        in_specs=[
            pl.BlockSpec((NB, DIN), lambda j, i: (i, 0)),
            pl.BlockSpec((DIN, DH // 2), lambda j, i: (0, j)),
            pl.BlockSpec((2, NB, 1), lambda j, i: (0, i, 0)),
        ],
        out_specs=pl.BlockSpec((1, NB, DH // 2), lambda j, i: (j, i, 0)),
        out_shape=jax.ShapeDtypeStruct((2, N, DH // 2), jnp.float32),
    )(x, w1, draw2)


def _mid_body(g_ref, hs_ref, d_ref, b_ref, w_ref, out_ref):
    scale = lax.rsqrt(d_ref[0] + d_ref[1] + 1.0)
    a = jax.nn.relu((g_ref[...] + hs_ref[...]) * scale[None]
                    + b_ref[...].reshape(2, 1, DH // 2))
    h2 = (jnp.dot(a[0], w_ref[0], preferred_element_type=jnp.float32)
          + jnp.dot(a[1], w_ref[1], preferred_element_type=jnp.float32))
    out_ref[...] = h2 * scale


def _mid(g1, hs1, draw2, b1r, w2r):
    return pl.pallas_call(
        _mid_body,
        grid=(NBLK,),
        in_specs=[
            pl.BlockSpec((2, NB, DH // 2), lambda i: (0, i, 0)),
            pl.BlockSpec((2, NB, DH // 2), lambda i: (0, i, 0)),
            pl.BlockSpec((2, NB, 1), lambda i: (0, i, 0)),
            pl.BlockSpec((2, DH // 2), lambda i: (0, 0)),
            pl.BlockSpec((2, DH // 2, DOUT), lambda i: (0, 0, 0)),
        ],
        out_specs=pl.BlockSpec((NB, DOUT), lambda i: (i, 0)),
        out_shape=jax.ShapeDtypeStruct((N, DOUT), jnp.float32),
    )(g1, hs1, draw2, b1r, w2r)


def _out_body(g_ref, hs_ref, d_ref, b_ref, out_ref):
    scale = lax.rsqrt(d_ref[0] + d_ref[1] + 1.0)
    zf = (g_ref[0] + g_ref[1] + hs_ref[...]) * scale + b_ref[...]
    m = jnp.max(zf, axis=-1, keepdims=True)
    lse = jnp.log(jnp.sum(jnp.exp(zf - m), axis=-1, keepdims=True)) + m
    out_ref[...] = zf - lse


def _outk(g2, hs2, draw2, b2r):
    return pl.pallas_call(
        _out_body,
        grid=(NBLK,),
        in_specs=[
            pl.BlockSpec((2, NB, DOUT), lambda i: (0, i, 0)),
            pl.BlockSpec((NB, DOUT), lambda i: (i, 0)),
            pl.BlockSpec((2, NB, 1), lambda i: (0, i, 0)),
            pl.BlockSpec((1, DOUT), lambda i: (0, 0)),
        ],
        out_specs=pl.BlockSpec((NB, DOUT), lambda i: (i, 0)),
        out_shape=jax.ShapeDtypeStruct((N, DOUT), jnp.float32),
    )(g2, hs2, draw2, b2r)


def kernel(x, edge_index, W1, b1, W2, b2):
    src2 = edge_index[0].reshape(EROWS, EC)
    dst2 = edge_index[1].reshape(EROWS, EC)

    draw = _deg_kernel(dst2)
    draw2 = draw.reshape(2, N, 1)

    hs1 = _mm1(x, W1, draw2)
    g1 = _agg_l1(hs1, src2, dst2)
    hs2 = _mid(g1, hs1, draw2, b1.reshape(2, DH // 2),
               W2.reshape(2, DH // 2, DOUT))
    g2 = _agg_l2(hs2, src2, dst2)
    return _outk(g2, hs2, draw2, b2.reshape(1, DOUT))
